# Initial kernel scaffold; baseline (speedup 1.0000x reference)
import jax
import jax.numpy as jnp
from jax.experimental import pallas as pl


def kernel(h_node, subgraph_batch, subgraph_idx_batch, W0, b0, Ws0, bs0, W1, b1, Ws1, bs1, Wf1, bf1, Wf2, bf2):
    raise NotImplementedError("write your pallas kernel here")


# speedup vs baseline: 6.2364x; 6.2364x over previous
import functools

import jax
import jax.numpy as jnp
from jax import lax
from jax.experimental import pallas as pl
from jax.experimental.pallas import tpu as pltpu
from jax.experimental.pallas import tpu_sc as plsc

N_CORES = 2
N_SUBCORES = 16
NW = N_CORES * N_SUBCORES
CHUNK = 80
N_GRAPHS = 64


def _sc_segment_sums(h_node, idx3, n_seg):
    n, d = h_node.shape
    n_chunks = idx3.shape[1]
    rows_per_w = n_chunks * CHUNK
    assert rows_per_w * NW == n
    seg_per_sub = n_seg // N_SUBCORES

    mesh = plsc.VectorSubcoreMesh(
        core_axis_name="c", subcore_axis_name="s",
        num_cores=N_CORES, num_subcores=N_SUBCORES)

    @functools.partial(
        pl.kernel,
        out_type=[
            jax.ShapeDtypeStruct((N_CORES, n_seg, d), jnp.float32),
            jax.ShapeDtypeStruct((N_CORES * n_seg,), jnp.float32),
        ],
        mesh=mesh,
        scratch_types=[
            pltpu.VMEM((n_chunks, CHUNK), jnp.int32),
            pltpu.VMEM((CHUNK, d), jnp.float32),
            pltpu.VMEM((128,), jnp.float32),
            pltpu.VMEM((256,), jnp.float32),
            pltpu.VMEM_SHARED((n_seg, d), jnp.float32),
            pltpu.VMEM_SHARED((n_seg,), jnp.float32),
        ],
    )
    def seg_kernel(h_hbm, idx_hbm, psum_hbm, pcnt_hbm,
                   idx_v, rows_v, ones_v, czero_v, acc_sh, cnt_sh):
        cid = lax.axis_index("c")
        sid = lax.axis_index("s")
        wid = cid * N_SUBCORES + sid

        one16 = jnp.ones((16,), jnp.float32)
        zero16 = jnp.zeros((16,), jnp.float32)
        for k in range(8):
            ones_v[pl.ds(k * 16, 16)] = one16
        for k in range(16):
            czero_v[pl.ds(k * 16, 16)] = zero16

        def zero_row(i, carry):
            for k in range(d // 16):
                rows_v[i, pl.ds(k * 16, 16)] = zero16
            return carry
        lax.fori_loop(0, CHUNK, zero_row, 0)

        base_seg = sid * seg_per_sub
        done = 0
        while done < seg_per_sub:
            step = min(CHUNK, seg_per_sub - done)
            pltpu.sync_copy(rows_v.at[pl.ds(0, step)],
                            acc_sh.at[pl.ds(base_seg + done, step)])
            done += step
        pltpu.sync_copy(czero_v.at[pl.ds(0, seg_per_sub)],
                        cnt_sh.at[pl.ds(base_seg, seg_per_sub)])
        plsc.subcore_barrier()

        pltpu.sync_copy(idx_hbm.at[wid], idx_v)
        row_base = wid * rows_per_w

        def body(j, carry):
            pltpu.sync_copy(h_hbm.at[pl.ds(row_base + j * CHUNK, CHUNK)], rows_v)
            pltpu.sync_copy(rows_v, acc_sh.at[idx_v.at[j]], add=True)
            pltpu.sync_copy(ones_v.at[pl.ds(0, CHUNK)],
                            cnt_sh.at[idx_v.at[j]], add=True)
            return carry
        lax.fori_loop(0, n_chunks, body, 0)
        plsc.subcore_barrier()

        pltpu.sync_copy(acc_sh.at[pl.ds(base_seg, seg_per_sub)],
                        psum_hbm.at[cid, pl.ds(base_seg, seg_per_sub)])
        pltpu.sync_copy(cnt_sh.at[pl.ds(base_seg, seg_per_sub)],
                        czero_v.at[pl.ds(0, seg_per_sub)])
        pltpu.sync_copy(czero_v.at[pl.ds(0, seg_per_sub)],
                        pcnt_hbm.at[pl.ds(cid * n_seg + base_seg, seg_per_sub)])

    return seg_kernel(h_node, idx3)


def _tc_dense(psum, pcnt, gid_col, gid_row,
              W0, b0, Ws0, bs0, W1, b1, Ws1, bs1, Wf1, bf1, Wf2, bf2):
    n_seg = psum.shape[1]
    n_tasks = Wf2.shape[1]

    def body(psum_ref, pcnt_ref, gc_ref, gr_ref,
             w0_ref, b0_ref, ws0_ref, bs0_ref,
             w1_ref, b1_ref, ws1_ref, bs1_ref,
             wf1_ref, bf1_ref, wf2_ref, bf2_ref, out_ref):
        f32 = jnp.float32
        sums = psum_ref[0] + psum_ref[1]
        cnt = pcnt_ref[0] + pcnt_ref[1]
        h = sums / jnp.maximum(cnt, 1.0)

        gc = gc_ref[...]
        gr = gr_ref[...]
        onehot = (gc == lax.broadcasted_iota(jnp.int32, (n_seg, N_GRAPHS), 1)
                  ).astype(f32)
        onehot_t = (lax.broadcasted_iota(jnp.int32, (N_GRAPHS, n_seg), 0) == gr
                    ).astype(f32)
        gcnt = jnp.dot(onehot_t, jnp.ones((n_seg, 1), f32),
                       preferred_element_type=f32, precision=lax.Precision.HIGHEST)
        inv_gcnt = 1.0 / jnp.maximum(gcnt, 1.0)

        def ds_layer(h, w_ref, b_ref, ws_ref, bs_ref):
            x1 = jnp.dot(h, w_ref[...], preferred_element_type=f32, precision=lax.Precision.HIGHEST) + b_ref[...]
            pooled = jnp.dot(onehot_t, h, preferred_element_type=f32, precision=lax.Precision.HIGHEST) * inv_gcnt
            x2 = jnp.dot(pooled, ws_ref[...], preferred_element_type=f32, precision=lax.Precision.HIGHEST) + bs_ref[...]
            x2b = jnp.dot(onehot, x2, preferred_element_type=f32, precision=lax.Precision.HIGHEST)
            v = x1 + x2b
            return jnp.where(v > 0, v, jnp.exp(jnp.minimum(v, 0.0)) - 1.0)

        h = ds_layer(h, w0_ref, b0_ref, ws0_ref, bs0_ref)
        h = ds_layer(h, w1_ref, b1_ref, ws1_ref, bs1_ref)

        hg = jnp.dot(onehot_t, h, preferred_element_type=f32, precision=lax.Precision.HIGHEST) * inv_gcnt
        t = jnp.maximum(jnp.dot(hg, wf1_ref[...], preferred_element_type=f32, precision=lax.Precision.HIGHEST)
                        + bf1_ref[...], 0.0)
        out_ref[...] = (jnp.dot(t, wf2_ref[...], preferred_element_type=f32, precision=lax.Precision.HIGHEST)
                        + bf2_ref[...])

    return pl.pallas_call(
        body,
        out_shape=jax.ShapeDtypeStruct((N_GRAPHS, n_tasks), jnp.float32),
    )(psum, pcnt, gid_col, gid_row,
      W0, b0, Ws0, bs0, W1, b1, Ws1, bs1, Wf1, bf1, Wf2, bf2)


def kernel(h_node, subgraph_batch, subgraph_idx_batch,
           W0, b0, Ws0, bs0, W1, b1, Ws1, bs1, Wf1, bf1, Wf2, bf2):
    n = h_node.shape[0]
    n_seg = subgraph_idx_batch.shape[0]
    n_chunks = n // (NW * CHUNK)
    idx3 = subgraph_batch.reshape(NW, n_chunks, CHUNK)

    psum, pcnt = _sc_segment_sums(h_node, idx3, n_seg)

    return _tc_dense(
        psum, pcnt.reshape(N_CORES, n_seg, 1),
        subgraph_idx_batch.reshape(n_seg, 1),
        subgraph_idx_batch.reshape(1, n_seg),
        W0, b0.reshape(1, -1), Ws0, bs0.reshape(1, -1),
        W1, b1.reshape(1, -1), Ws1, bs1.reshape(1, -1),
        Wf1, bf1.reshape(1, -1), Wf2, bf2.reshape(1, -1))


# speedup vs baseline: 7.9397x; 1.2731x over previous
import functools

import jax
import jax.numpy as jnp
from jax import lax
from jax.experimental import pallas as pl
from jax.experimental.pallas import tpu as pltpu
from jax.experimental.pallas import tpu_sc as plsc

N_CORES = 2
N_SUBCORES = 16
NW = N_CORES * N_SUBCORES
CHUNK = 80
N_GRAPHS = 64


def _sc_segment_sums(h_node, idx3, n_seg):
    n, d = h_node.shape
    n_chunks = idx3.shape[1]
    rows_per_w = n_chunks * CHUNK
    assert rows_per_w * NW == n
    seg_per_sub = n_seg // N_SUBCORES

    mesh = plsc.VectorSubcoreMesh(
        core_axis_name="c", subcore_axis_name="s",
        num_cores=N_CORES, num_subcores=N_SUBCORES)

    @functools.partial(
        pl.kernel,
        out_type=[
            jax.ShapeDtypeStruct((N_CORES, n_seg, d), jnp.float32),
            jax.ShapeDtypeStruct((N_CORES * n_seg,), jnp.float32),
        ],
        mesh=mesh,
        scratch_types=[
            pltpu.VMEM((n_chunks, CHUNK), jnp.int32),
            pltpu.VMEM((CHUNK, d), jnp.float32),
            pltpu.VMEM((CHUNK, d), jnp.float32),
            pltpu.VMEM((128,), jnp.float32),
            pltpu.VMEM((256,), jnp.float32),
            pltpu.VMEM_SHARED((n_seg, d), jnp.float32),
            pltpu.VMEM_SHARED((n_seg,), jnp.float32),
            pltpu.SemaphoreType.DMA,
            pltpu.SemaphoreType.DMA,
            pltpu.SemaphoreType.DMA,
            pltpu.SemaphoreType.DMA,
        ],
    )
    def seg_kernel(h_hbm, idx_hbm, psum_hbm, pcnt_hbm,
                   idx_v, rows_v, rows2_v, ones_v, czero_v, acc_sh, cnt_sh,
                   gsem_a, gsem_b, ssem_a, ssem_b):
        cid = lax.axis_index("c")
        sid = lax.axis_index("s")
        wid = cid * N_SUBCORES + sid

        one16 = jnp.ones((16,), jnp.float32)
        zero16 = jnp.zeros((16,), jnp.float32)
        for k in range(8):
            ones_v[pl.ds(k * 16, 16)] = one16
        for k in range(16):
            czero_v[pl.ds(k * 16, 16)] = zero16

        def zero_row(i, carry):
            for k in range(d // 16):
                rows_v[i, pl.ds(k * 16, 16)] = zero16
            return carry
        lax.fori_loop(0, CHUNK, zero_row, 0)

        base_seg = sid * seg_per_sub
        done = 0
        while done < seg_per_sub:
            step = min(CHUNK, seg_per_sub - done)
            pltpu.sync_copy(rows_v.at[pl.ds(0, step)],
                            acc_sh.at[pl.ds(base_seg + done, step)])
            done += step
        pltpu.sync_copy(czero_v.at[pl.ds(0, seg_per_sub)],
                        cnt_sh.at[pl.ds(base_seg, seg_per_sub)])
        plsc.subcore_barrier()

        pltpu.sync_copy(idx_hbm.at[wid], idx_v)
        row_base = wid * rows_per_w
        ones_c = ones_v.at[pl.ds(0, CHUNK)]
        last = n_chunks - 1

        def gather(j, buf, sem):
            return pltpu.async_copy(
                h_hbm.at[pl.ds(row_base + j * CHUNK, CHUNK)], buf, sem)

        def gather_wait(buf, sem):
            pltpu.make_async_copy(h_hbm.at[pl.ds(0, CHUNK)], buf, sem).wait()

        gather(0, rows_v, gsem_a)
        gather(1, rows2_v, gsem_b)

        def body(i, carry):
            j0 = 2 * i
            gather_wait(rows_v, gsem_a)
            a0 = pltpu.async_copy(rows_v, acc_sh.at[idx_v.at[j0]], ssem_a,
                                  add=True)
            c0 = pltpu.async_copy(ones_c, cnt_sh.at[idx_v.at[j0]], ssem_a,
                                  add=True)
            gather_wait(rows2_v, gsem_b)
            a1 = pltpu.async_copy(rows2_v, acc_sh.at[idx_v.at[j0 + 1]], ssem_b,
                                  add=True)
            c1 = pltpu.async_copy(ones_c, cnt_sh.at[idx_v.at[j0 + 1]], ssem_b,
                                  add=True)
            a0.wait()
            c0.wait()
            gather(jnp.minimum(j0 + 2, last), rows_v, gsem_a)
            a1.wait()
            c1.wait()
            gather(jnp.minimum(j0 + 3, last), rows2_v, gsem_b)
            return carry
        lax.fori_loop(0, (n_chunks - 1) // 2, body, 0)

        gather_wait(rows_v, gsem_a)
        pltpu.sync_copy(rows_v, acc_sh.at[idx_v.at[last]], add=True)
        pltpu.sync_copy(ones_c, cnt_sh.at[idx_v.at[last]], add=True)
        gather_wait(rows2_v, gsem_b)
        plsc.subcore_barrier()

        pltpu.sync_copy(acc_sh.at[pl.ds(base_seg, seg_per_sub)],
                        psum_hbm.at[cid, pl.ds(base_seg, seg_per_sub)])
        pltpu.sync_copy(cnt_sh.at[pl.ds(base_seg, seg_per_sub)],
                        czero_v.at[pl.ds(0, seg_per_sub)])
        pltpu.sync_copy(czero_v.at[pl.ds(0, seg_per_sub)],
                        pcnt_hbm.at[pl.ds(cid * n_seg + base_seg, seg_per_sub)])

    return seg_kernel(h_node, idx3)


def _tc_dense(psum, pcnt, gid_col, gid_row,
              W0, b0, Ws0, bs0, W1, b1, Ws1, bs1, Wf1, bf1, Wf2, bf2):
    n_seg = psum.shape[1]
    n_tasks = Wf2.shape[1]

    def body(psum_ref, pcnt_ref, gc_ref, gr_ref,
             w0_ref, b0_ref, ws0_ref, bs0_ref,
             w1_ref, b1_ref, ws1_ref, bs1_ref,
             wf1_ref, bf1_ref, wf2_ref, bf2_ref, out_ref):
        f32 = jnp.float32
        sums = psum_ref[0] + psum_ref[1]
        cnt = pcnt_ref[0] + pcnt_ref[1]
        h = sums / jnp.maximum(cnt, 1.0)

        gc = gc_ref[...]
        gr = gr_ref[...]
        onehot = (gc == lax.broadcasted_iota(jnp.int32, (n_seg, N_GRAPHS), 1)
                  ).astype(f32)
        onehot_t = (lax.broadcasted_iota(jnp.int32, (N_GRAPHS, n_seg), 0) == gr
                    ).astype(f32)
        gcnt = jnp.dot(onehot_t, jnp.ones((n_seg, 1), f32),
                       preferred_element_type=f32, precision=lax.Precision.HIGHEST)
        inv_gcnt = 1.0 / jnp.maximum(gcnt, 1.0)

        def ds_layer(h, w_ref, b_ref, ws_ref, bs_ref):
            x1 = jnp.dot(h, w_ref[...], preferred_element_type=f32, precision=lax.Precision.HIGHEST) + b_ref[...]
            pooled = jnp.dot(onehot_t, h, preferred_element_type=f32, precision=lax.Precision.HIGHEST) * inv_gcnt
            x2 = jnp.dot(pooled, ws_ref[...], preferred_element_type=f32, precision=lax.Precision.HIGHEST) + bs_ref[...]
            x2b = jnp.dot(onehot, x2, preferred_element_type=f32, precision=lax.Precision.HIGHEST)
            v = x1 + x2b
            return jnp.where(v > 0, v, jnp.exp(jnp.minimum(v, 0.0)) - 1.0)

        h = ds_layer(h, w0_ref, b0_ref, ws0_ref, bs0_ref)
        h = ds_layer(h, w1_ref, b1_ref, ws1_ref, bs1_ref)

        hg = jnp.dot(onehot_t, h, preferred_element_type=f32, precision=lax.Precision.HIGHEST) * inv_gcnt
        t = jnp.maximum(jnp.dot(hg, wf1_ref[...], preferred_element_type=f32, precision=lax.Precision.HIGHEST)
                        + bf1_ref[...], 0.0)
        out_ref[...] = (jnp.dot(t, wf2_ref[...], preferred_element_type=f32, precision=lax.Precision.HIGHEST)
                        + bf2_ref[...])

    return pl.pallas_call(
        body,
        out_shape=jax.ShapeDtypeStruct((N_GRAPHS, n_tasks), jnp.float32),
    )(psum, pcnt, gid_col, gid_row,
      W0, b0, Ws0, bs0, W1, b1, Ws1, bs1, Wf1, bf1, Wf2, bf2)


def kernel(h_node, subgraph_batch, subgraph_idx_batch,
           W0, b0, Ws0, bs0, W1, b1, Ws1, bs1, Wf1, bf1, Wf2, bf2):
    n = h_node.shape[0]
    n_seg = subgraph_idx_batch.shape[0]
    n_chunks = n // (NW * CHUNK)
    idx3 = subgraph_batch.reshape(NW, n_chunks, CHUNK)

    psum, pcnt = _sc_segment_sums(h_node, idx3, n_seg)

    return _tc_dense(
        psum, pcnt.reshape(N_CORES, n_seg, 1),
        subgraph_idx_batch.reshape(n_seg, 1),
        subgraph_idx_batch.reshape(1, n_seg),
        W0, b0.reshape(1, -1), Ws0, bs0.reshape(1, -1),
        W1, b1.reshape(1, -1), Ws1, bs1.reshape(1, -1),
        Wf1, bf1.reshape(1, -1), Wf2, bf2.reshape(1, -1))


# speedup vs baseline: 10.3906x; 1.3087x over previous
import functools

import jax
import jax.numpy as jnp
from jax import lax
from jax.experimental import pallas as pl
from jax.experimental.pallas import tpu as pltpu
from jax.experimental.pallas import tpu_sc as plsc

N_CORES = 2
N_SUBCORES = 16
NW = N_CORES * N_SUBCORES
CHUNK = 80
N_GRAPHS = 64


def _sc_segment_sums(h_node, idx3, n_seg):
    n, d = h_node.shape
    n_chunks = idx3.shape[1]
    rows_per_w = n_chunks * CHUNK
    assert rows_per_w * NW == n
    seg_per_sub = n_seg // N_SUBCORES

    mesh = plsc.VectorSubcoreMesh(
        core_axis_name="c", subcore_axis_name="s",
        num_cores=N_CORES, num_subcores=N_SUBCORES)

    @functools.partial(
        pl.kernel,
        out_type=[
            jax.ShapeDtypeStruct((N_CORES, n_seg, d), jnp.float32),
            jax.ShapeDtypeStruct((N_CORES * n_seg,), jnp.float32),
        ],
        mesh=mesh,
        scratch_types=[
            pltpu.VMEM((n_chunks, CHUNK), jnp.int32),
            [pltpu.VMEM((CHUNK, d), jnp.float32) for _ in range(4)],
            pltpu.VMEM((128,), jnp.float32),
            pltpu.VMEM((256,), jnp.float32),
            pltpu.VMEM_SHARED((n_seg, d), jnp.float32),
            pltpu.VMEM_SHARED((n_seg,), jnp.float32),
            [pltpu.SemaphoreType.DMA for _ in range(4)],
            [pltpu.SemaphoreType.DMA for _ in range(4)],
        ],
    )
    def seg_kernel(h_hbm, idx_hbm, psum_hbm, pcnt_hbm,
                   idx_v, bufs, ones_v, czero_v, acc_sh, cnt_sh,
                   gsems, ssems):
        rows_v = bufs[0]
        cid = lax.axis_index("c")
        sid = lax.axis_index("s")
        wid = cid * N_SUBCORES + sid

        one16 = jnp.ones((16,), jnp.float32)
        zero16 = jnp.zeros((16,), jnp.float32)
        for k in range(8):
            ones_v[pl.ds(k * 16, 16)] = one16
        for k in range(16):
            czero_v[pl.ds(k * 16, 16)] = zero16

        def zero_row(i, carry):
            for k in range(d // 16):
                rows_v[i, pl.ds(k * 16, 16)] = zero16
            return carry
        lax.fori_loop(0, CHUNK, zero_row, 0)

        base_seg = sid * seg_per_sub
        done = 0
        while done < seg_per_sub:
            step = min(CHUNK, seg_per_sub - done)
            pltpu.sync_copy(rows_v.at[pl.ds(0, step)],
                            acc_sh.at[pl.ds(base_seg + done, step)])
            done += step
        pltpu.sync_copy(czero_v.at[pl.ds(0, seg_per_sub)],
                        cnt_sh.at[pl.ds(base_seg, seg_per_sub)])
        plsc.subcore_barrier()

        pltpu.sync_copy(idx_hbm.at[wid], idx_v)
        row_base = wid * rows_per_w
        ones_c = ones_v.at[pl.ds(0, CHUNK)]
        last = n_chunks - 1

        def gather(j, b):
            pltpu.async_copy(
                h_hbm.at[pl.ds(row_base + j * CHUNK, CHUNK)], bufs[b], gsems[b])

        def gather_wait(b):
            pltpu.make_async_copy(h_hbm.at[pl.ds(0, CHUNK)], bufs[b],
                                  gsems[b]).wait()

        for b in range(4):
            gather(b, b)

        def scat(j, b):
            a = pltpu.async_copy(bufs[b], acc_sh.at[idx_v.at[j]], ssems[b],
                                 add=True)
            c = pltpu.async_copy(ones_c, cnt_sh.at[idx_v.at[j]], ssems[b],
                                 add=True)
            return a, c

        def body(i, carry):
            j0 = 4 * i
            pend = {}
            gather_wait(0)
            pend[0] = scat(j0, 0)
            gather_wait(1)
            pend[1] = scat(j0 + 1, 1)
            for b in range(2, 4):
                d = b - 2
                pend[d][0].wait()
                pend[d][1].wait()
                gather(jnp.minimum(j0 + d + 4, last), d)
                gather_wait(b)
                pend[b] = scat(j0 + b, b)
            for b in range(2, 4):
                pend[b][0].wait()
                pend[b][1].wait()
                gather(jnp.minimum(j0 + b + 4, last), b)
            return carry
        lax.fori_loop(0, (n_chunks - 1) // 4, body, 0)

        gather_wait(0)
        pltpu.sync_copy(bufs[0], acc_sh.at[idx_v.at[last]], add=True)
        pltpu.sync_copy(ones_c, cnt_sh.at[idx_v.at[last]], add=True)
        for b in range(1, 4):
            gather_wait(b)
        plsc.subcore_barrier()

        pltpu.sync_copy(acc_sh.at[pl.ds(base_seg, seg_per_sub)],
                        psum_hbm.at[cid, pl.ds(base_seg, seg_per_sub)])
        pltpu.sync_copy(cnt_sh.at[pl.ds(base_seg, seg_per_sub)],
                        czero_v.at[pl.ds(0, seg_per_sub)])
        pltpu.sync_copy(czero_v.at[pl.ds(0, seg_per_sub)],
                        pcnt_hbm.at[pl.ds(cid * n_seg + base_seg, seg_per_sub)])

    return seg_kernel(h_node, idx3)


def _tc_dense(psum, pcnt, gid_col, gid_row,
              W0, b0, Ws0, bs0, W1, b1, Ws1, bs1, Wf1, bf1, Wf2, bf2):
    n_seg = psum.shape[1]
    n_tasks = Wf2.shape[1]

    def body(psum_ref, pcnt_ref, gc_ref, gr_ref,
             w0_ref, b0_ref, ws0_ref, bs0_ref,
             w1_ref, b1_ref, ws1_ref, bs1_ref,
             wf1_ref, bf1_ref, wf2_ref, bf2_ref, out_ref):
        f32 = jnp.float32
        sums = psum_ref[0] + psum_ref[1]
        cnt = pcnt_ref[0] + pcnt_ref[1]
        h = sums / jnp.maximum(cnt, 1.0)

        gc = gc_ref[...]
        gr = gr_ref[...]
        onehot = (gc == lax.broadcasted_iota(jnp.int32, (n_seg, N_GRAPHS), 1)
                  ).astype(f32)
        onehot_t = (lax.broadcasted_iota(jnp.int32, (N_GRAPHS, n_seg), 0) == gr
                    ).astype(f32)
        gcnt = jnp.dot(onehot_t, jnp.ones((n_seg, 1), f32),
                       preferred_element_type=f32, precision=lax.Precision.HIGHEST)
        inv_gcnt = 1.0 / jnp.maximum(gcnt, 1.0)

        def ds_layer(h, w_ref, b_ref, ws_ref, bs_ref):
            x1 = jnp.dot(h, w_ref[...], preferred_element_type=f32, precision=lax.Precision.HIGHEST) + b_ref[...]
            pooled = jnp.dot(onehot_t, h, preferred_element_type=f32, precision=lax.Precision.HIGHEST) * inv_gcnt
            x2 = jnp.dot(pooled, ws_ref[...], preferred_element_type=f32, precision=lax.Precision.HIGHEST) + bs_ref[...]
            x2b = jnp.dot(onehot, x2, preferred_element_type=f32, precision=lax.Precision.HIGHEST)
            v = x1 + x2b
            return jnp.where(v > 0, v, jnp.exp(jnp.minimum(v, 0.0)) - 1.0)

        h = ds_layer(h, w0_ref, b0_ref, ws0_ref, bs0_ref)
        h = ds_layer(h, w1_ref, b1_ref, ws1_ref, bs1_ref)

        hg = jnp.dot(onehot_t, h, preferred_element_type=f32, precision=lax.Precision.HIGHEST) * inv_gcnt
        t = jnp.maximum(jnp.dot(hg, wf1_ref[...], preferred_element_type=f32, precision=lax.Precision.HIGHEST)
                        + bf1_ref[...], 0.0)
        out_ref[...] = (jnp.dot(t, wf2_ref[...], preferred_element_type=f32, precision=lax.Precision.HIGHEST)
                        + bf2_ref[...])

    return pl.pallas_call(
        body,
        out_shape=jax.ShapeDtypeStruct((N_GRAPHS, n_tasks), jnp.float32),
    )(psum, pcnt, gid_col, gid_row,
      W0, b0, Ws0, bs0, W1, b1, Ws1, bs1, Wf1, bf1, Wf2, bf2)


def kernel(h_node, subgraph_batch, subgraph_idx_batch,
           W0, b0, Ws0, bs0, W1, b1, Ws1, bs1, Wf1, bf1, Wf2, bf2):
    n = h_node.shape[0]
    n_seg = subgraph_idx_batch.shape[0]
    n_chunks = n // (NW * CHUNK)
    idx3 = subgraph_batch.reshape(NW, n_chunks, CHUNK)

    psum, pcnt = _sc_segment_sums(h_node, idx3, n_seg)

    return _tc_dense(
        psum, pcnt.reshape(N_CORES, n_seg, 1),
        subgraph_idx_batch.reshape(n_seg, 1),
        subgraph_idx_batch.reshape(1, n_seg),
        W0, b0.reshape(1, -1), Ws0, bs0.reshape(1, -1),
        W1, b1.reshape(1, -1), Ws1, bs1.reshape(1, -1),
        Wf1, bf1.reshape(1, -1), Wf2, bf2.reshape(1, -1))


# speedup vs baseline: 11.4177x; 1.0988x over previous
import functools

import jax
import jax.numpy as jnp
from jax import lax
from jax.experimental import pallas as pl
from jax.experimental.pallas import tpu as pltpu
from jax.experimental.pallas import tpu_sc as plsc

N_CORES = 2
N_SUBCORES = 16
NW = N_CORES * N_SUBCORES
CHUNK = 80
N_GRAPHS = 64


def _sc_segment_sums(h_node, idx3, n_seg):
    n, d = h_node.shape
    n_chunks = idx3.shape[1]
    rows_per_w = n_chunks * CHUNK
    assert rows_per_w * NW == n
    seg_per_sub = n_seg // N_SUBCORES

    mesh = plsc.VectorSubcoreMesh(
        core_axis_name="c", subcore_axis_name="s",
        num_cores=N_CORES, num_subcores=N_SUBCORES)

    @functools.partial(
        pl.kernel,
        out_type=[
            jax.ShapeDtypeStruct((N_CORES, n_seg, d), jnp.float32),
            jax.ShapeDtypeStruct((N_CORES * n_seg,), jnp.float32),
        ],
        mesh=mesh,
        scratch_types=[
            pltpu.VMEM((n_chunks, CHUNK), jnp.int32),
            [pltpu.VMEM((CHUNK, d), jnp.float32) for _ in range(4)],
            pltpu.VMEM((CHUNK, d), jnp.float32),
            pltpu.VMEM((128,), jnp.float32),
            pltpu.VMEM((256,), jnp.float32),
            pltpu.VMEM_SHARED((n_seg, d), jnp.float32),
            pltpu.VMEM_SHARED((n_seg,), jnp.float32),
            [pltpu.SemaphoreType.DMA for _ in range(4)],
            [pltpu.SemaphoreType.DMA for _ in range(4)],
            pltpu.SemaphoreType.DMA,
        ],
    )
    def seg_kernel(h_hbm, idx_hbm, psum_hbm, pcnt_hbm,
                   idx_v, bufs, zrow_v, ones_v, czero_v, acc_sh, cnt_sh,
                   gsems, ssems, isem):
        cid = lax.axis_index("c")
        sid = lax.axis_index("s")
        wid = cid * N_SUBCORES + sid
        row_base = wid * rows_per_w
        last = n_chunks - 1

        def gather(j, b):
            pltpu.async_copy(
                h_hbm.at[pl.ds(row_base + j * CHUNK, CHUNK)], bufs[b], gsems[b])

        def gather_wait(b):
            pltpu.make_async_copy(h_hbm.at[pl.ds(0, CHUNK)], bufs[b],
                                  gsems[b]).wait()

        for b in range(4):
            gather(b, b)
        idx_load = pltpu.async_copy(idx_hbm.at[wid], idx_v, isem)

        one16 = jnp.ones((16,), jnp.float32)
        zero16 = jnp.zeros((16,), jnp.float32)
        for k in range(8):
            ones_v[pl.ds(k * 16, 16)] = one16
        for k in range(16):
            czero_v[pl.ds(k * 16, 16)] = zero16

        def zero_row(i, carry):
            for k in range(d // 16):
                zrow_v[i, pl.ds(k * 16, 16)] = zero16
            return carry
        lax.fori_loop(0, CHUNK, zero_row, 0)

        base_seg = sid * seg_per_sub
        done = 0
        while done < seg_per_sub:
            step = min(CHUNK, seg_per_sub - done)
            pltpu.sync_copy(zrow_v.at[pl.ds(0, step)],
                            acc_sh.at[pl.ds(base_seg + done, step)])
            done += step
        pltpu.sync_copy(czero_v.at[pl.ds(0, seg_per_sub)],
                        cnt_sh.at[pl.ds(base_seg, seg_per_sub)])
        plsc.subcore_barrier()
        idx_load.wait()

        ones_c = ones_v.at[pl.ds(0, CHUNK)]

        def scat(j, b):
            a = pltpu.async_copy(bufs[b], acc_sh.at[idx_v.at[j]], ssems[b],
                                 add=True)
            c = pltpu.async_copy(ones_c, cnt_sh.at[idx_v.at[j]], ssems[b],
                                 add=True)
            return a, c

        def body(i, carry):
            j0 = 4 * i
            pend = {}
            gather_wait(0)
            pend[0] = scat(j0, 0)
            gather_wait(1)
            pend[1] = scat(j0 + 1, 1)
            for b in range(2, 4):
                d = b - 2
                for w in pend[d]:
                    w.wait()
                gather(jnp.minimum(j0 + d + 4, last), d)
                gather_wait(b)
                pend[b] = scat(j0 + b, b)
            for b in range(2, 4):
                for w in pend[b]:
                    w.wait()
                gather(jnp.minimum(j0 + b + 4, last), b)
            return carry
        lax.fori_loop(0, (n_chunks - 1) // 4, body, 0)

        gather_wait(0)
        pltpu.sync_copy(bufs[0], acc_sh.at[idx_v.at[last]], add=True)
        pltpu.sync_copy(ones_c, cnt_sh.at[idx_v.at[last]], add=True)
        for b in range(1, 4):
            gather_wait(b)
        plsc.subcore_barrier()

        pltpu.sync_copy(acc_sh.at[pl.ds(base_seg, seg_per_sub)],
                        psum_hbm.at[cid, pl.ds(base_seg, seg_per_sub)])
        pltpu.sync_copy(cnt_sh.at[pl.ds(base_seg, seg_per_sub)],
                        czero_v.at[pl.ds(0, seg_per_sub)])
        pltpu.sync_copy(czero_v.at[pl.ds(0, seg_per_sub)],
                        pcnt_hbm.at[pl.ds(cid * n_seg + base_seg, seg_per_sub)])

    return seg_kernel(h_node, idx3)


def _tc_dense(psum, pcnt, gid_col, gid_row,
              W0, b0, Ws0, bs0, W1, b1, Ws1, bs1, Wf1, bf1, Wf2, bf2):
    n_seg = psum.shape[1]
    n_tasks = Wf2.shape[1]

    def body(psum_ref, pcnt_ref, gc_ref, gr_ref,
             w0_ref, b0_ref, ws0_ref, bs0_ref,
             w1_ref, b1_ref, ws1_ref, bs1_ref,
             wf1_ref, bf1_ref, wf2_ref, bf2_ref, out_ref):
        f32 = jnp.float32
        sums = psum_ref[0] + psum_ref[1]
        cnt = pcnt_ref[0] + pcnt_ref[1]
        h = sums / jnp.maximum(cnt, 1.0)

        gc = gc_ref[...]
        gr = gr_ref[...]
        onehot = (gc == lax.broadcasted_iota(jnp.int32, (n_seg, N_GRAPHS), 1)
                  ).astype(f32)
        onehot_t = (lax.broadcasted_iota(jnp.int32, (N_GRAPHS, n_seg), 0) == gr
                    ).astype(f32)
        gcnt = jnp.dot(onehot_t, jnp.ones((n_seg, 1), f32),
                       preferred_element_type=f32)
        inv_gcnt = 1.0 / jnp.maximum(gcnt, 1.0)

        def ds_layer(h, w_ref, b_ref, ws_ref, bs_ref):
            x1 = jnp.dot(h, w_ref[...], preferred_element_type=f32) + b_ref[...]
            pooled = jnp.dot(onehot_t, h, preferred_element_type=f32) * inv_gcnt
            x2 = jnp.dot(pooled, ws_ref[...], preferred_element_type=f32) + bs_ref[...]
            x2b = jnp.dot(onehot, x2, preferred_element_type=f32)
            v = x1 + x2b
            return jnp.where(v > 0, v, jnp.exp(jnp.minimum(v, 0.0)) - 1.0)

        h = ds_layer(h, w0_ref, b0_ref, ws0_ref, bs0_ref)
        h = ds_layer(h, w1_ref, b1_ref, ws1_ref, bs1_ref)

        hg = jnp.dot(onehot_t, h, preferred_element_type=f32) * inv_gcnt
        t = jnp.maximum(jnp.dot(hg, wf1_ref[...], preferred_element_type=f32)
                        + bf1_ref[...], 0.0)
        out_ref[...] = (jnp.dot(t, wf2_ref[...], preferred_element_type=f32)
                        + bf2_ref[...])

    return pl.pallas_call(
        body,
        out_shape=jax.ShapeDtypeStruct((N_GRAPHS, n_tasks), jnp.float32),
    )(psum, pcnt, gid_col, gid_row,
      W0, b0, Ws0, bs0, W1, b1, Ws1, bs1, Wf1, bf1, Wf2, bf2)


def kernel(h_node, subgraph_batch, subgraph_idx_batch,
           W0, b0, Ws0, bs0, W1, b1, Ws1, bs1, Wf1, bf1, Wf2, bf2):
    n = h_node.shape[0]
    n_seg = subgraph_idx_batch.shape[0]
    n_chunks = n // (NW * CHUNK)
    idx3 = subgraph_batch.reshape(NW, n_chunks, CHUNK)

    psum, pcnt = _sc_segment_sums(h_node, idx3, n_seg)

    return _tc_dense(
        psum, pcnt.reshape(N_CORES, n_seg, 1),
        subgraph_idx_batch.reshape(n_seg, 1),
        subgraph_idx_batch.reshape(1, n_seg),
        W0, b0.reshape(1, -1), Ws0, bs0.reshape(1, -1),
        W1, b1.reshape(1, -1), Ws1, bs1.reshape(1, -1),
        Wf1, bf1.reshape(1, -1), Wf2, bf2.reshape(1, -1))
